# baseline (speedup 1.0000x reference)
import functools

import jax
import jax.numpy as jnp
from jax import lax
from jax.experimental import pallas as pl
from jax.experimental.pallas import tpu as pltpu
from jax.experimental.pallas import tpu_sc as plsc

B = 16384
D = 16
NREL = 64
CF_LAMBDA = 1e-05
KG_LAMBDA = 1e-05

_NC, _NS = 2, 16
NW = _NC * _NS
BPW = B // NW

@functools.cache
def _make_sc_gather():
    mesh = plsc.VectorSubcoreMesh(core_axis_name="c", subcore_axis_name="s")

    @functools.partial(
        pl.kernel,
        mesh=mesh,
        compiler_params=pltpu.CompilerParams(use_tc_tiling_on_sc=False),
        out_type=[jax.ShapeDtypeStruct((B, D), jnp.float32)] * 8,
        scratch_types=[
            pltpu.VMEM((BPW,), jnp.int32),
            pltpu.VMEM((BPW, D), jnp.float32),
            pltpu.SemaphoreType.DMA,
        ],
    )
    def _sc_gather(user_W, item_W, entity_W,
                   user_ids, ip_ids, ineg_ids, h_ids, pt_ids, nt_ids,
                   u_out, ip_out, ineg_out, ipk_out, inegk_out,
                   he_out, pt_out, nt_out,
                   idx_v, rows_v, sem):
        wid = lax.axis_index("s") * _NC + lax.axis_index("c")
        base = wid * BPW

        def load_idx(ids):
            pltpu.sync_copy(ids.at[pl.ds(base, BPW)], idx_v)

        def gather_to(tab, out):
            pltpu.async_copy(tab.at[idx_v], rows_v, sem).wait()
            pltpu.sync_copy(rows_v, out.at[pl.ds(base, BPW)])

        load_idx(user_ids)
        gather_to(user_W, u_out)
        load_idx(ip_ids)
        gather_to(item_W, ip_out)
        gather_to(entity_W, ipk_out)
        load_idx(ineg_ids)
        gather_to(item_W, ineg_out)
        gather_to(entity_W, inegk_out)
        load_idx(h_ids)
        gather_to(entity_W, he_out)
        load_idx(pt_ids)
        gather_to(entity_W, pt_out)
        load_idx(nt_ids)
        gather_to(entity_W, nt_out)

    return _sc_gather


def _tc_body(u_ref, ip_ref, ineg_ref, ipk_ref, inegk_ref,
             he_ref, pt_ref, nt_ref, r_ref, relW_ref, M2d_ref, out_ref):
    i = pl.program_id(0)

    @pl.when(i == 0)
    def _init():
        out_ref[...] = jnp.zeros((1, 1), jnp.float32)

    u = u_ref[...]
    pos_cf = ip_ref[...] + ipk_ref[...]
    neg_cf = ineg_ref[...] + inegk_ref[...]
    pos_s = jnp.sum(u * pos_cf, axis=1, keepdims=True)
    neg_s = jnp.sum(u * neg_cf, axis=1, keepdims=True)
    x = pos_s - neg_s
    sig = 1.0 / (1.0 + jnp.exp(-x))
    cf_term = -jnp.log(1e-10 + sig)
    l2_cf = 0.5 * (jnp.sum(u * u) + jnp.sum(pos_cf * pos_cf)
                   + jnp.sum(neg_cf * neg_cf))

    r_col = r_ref[...]
    k_row = lax.broadcasted_iota(jnp.int32, (1, NREL), 1)
    onehot = (r_col == k_row).astype(jnp.float32)
    Weff = jnp.dot(onehot, M2d_ref[...],
                   preferred_element_type=jnp.float32)
    re = jnp.dot(onehot, relW_ref[...],
                 preferred_element_type=jnp.float32)

    c1 = lax.broadcasted_iota(jnp.int32, (D, D * D), 1)
    d1 = lax.broadcasted_iota(jnp.int32, (D, D * D), 0)
    R = ((c1 // D) == d1).astype(jnp.float32)
    c2 = lax.broadcasted_iota(jnp.int32, (D * D, D), 0)
    j2 = lax.broadcasted_iota(jnp.int32, (D * D, D), 1)
    S = ((c2 % D) == j2).astype(jnp.float32)

    def proj(x_ref):
        xr = jnp.dot(x_ref[...], R, preferred_element_type=jnp.float32)
        return jnp.dot(xr * Weff, S, preferred_element_type=jnp.float32)

    rh = proj(he_ref)
    rpt = proj(pt_ref)
    rnt = proj(nt_ref)

    def normz(v):
        n = jnp.sqrt(jnp.sum(v * v, axis=1, keepdims=True))
        return v / jnp.maximum(n, 1e-12)

    re_n = normz(re)
    rh_n = normz(rh)
    rpt_n = normz(rpt)
    rnt_n = normz(rnt)

    dpos = rh_n + re_n - rpt_n
    dneg = rh_n + re_n - rnt_n
    pos_sc = jnp.sqrt(jnp.sum(dpos * dpos, axis=1, keepdims=True))
    neg_sc = jnp.sqrt(jnp.sum(dneg * dneg, axis=1, keepdims=True))
    kg_term = jnp.maximum(pos_sc - neg_sc + 1.0, 0.0)
    l2_kg = 0.5 * (jnp.sum(rh_n * rh_n) + jnp.sum(re_n * re_n)
                   + jnp.sum(rpt_n * rpt_n) + jnp.sum(rnt_n * rnt_n))

    block_total = (jnp.sum(cf_term) + CF_LAMBDA * l2_cf
                   + jnp.sum(kg_term) + KG_LAMBDA * l2_kg)
    out_ref[...] += jnp.reshape(block_total * (1.0 / B), (1, 1))


_BB = 2048


def _tc_call(u, ip, ineg, ipk, inegk, he, pt, nt, r2, rel_W, M2d,
             interpret=False):
    row_spec = pl.BlockSpec((_BB, D), lambda i: (i, 0))
    out = pl.pallas_call(
        _tc_body,
        grid=(B // _BB,),
        in_specs=[row_spec] * 8 + [
            pl.BlockSpec((_BB, 1), lambda i: (i, 0)),
            pl.BlockSpec((NREL, D), lambda i: (0, 0)),
            pl.BlockSpec((NREL, D * D), lambda i: (0, 0)),
        ],
        out_specs=pl.BlockSpec((1, 1), lambda i: (0, 0)),
        out_shape=jax.ShapeDtypeStruct((1, 1), jnp.float32),
        interpret=interpret,
    )(u, ip, ineg, ipk, inegk, he, pt, nt, r2, rel_W, M2d)
    return out[0, 0]


def kernel(user_W, item_W, entity_W, rel_W, trans_M,
           user_ids, item_pos_ids, item_neg_ids, h, r, pos_t, neg_t,
           is_train=1):
    i32 = lambda a: a.astype(jnp.int32)
    u, ip, ineg, ipk, inegk, he, pt, nt = _make_sc_gather()(
        user_W, item_W, entity_W,
        i32(user_ids), i32(item_pos_ids), i32(item_neg_ids),
        i32(h), i32(pos_t), i32(neg_t))
    M2d = trans_M.reshape(NREL, D * D)
    r2 = i32(r).reshape(B, 1)
    return _tc_call(u, ip, ineg, ipk, inegk, he, pt, nt, r2, rel_W, M2d)
